# baseline (speedup 1.0000x reference)
import jax
import jax.numpy as jnp
import numpy as np
from jax.experimental import pallas as pl

_B, _K, _L = 16, 100, 200
_TIME = 128
_FEAT = 16
_C = _TIME + _FEAT
_KL = _K * _L


def _body(tp_ref, div2_ref, off_ref, ett_ref, out_ref):
    pos = tp_ref[0, 0, :]
    angle = div2_ref[...] * pos[None, :] + off_ref[...]
    pe = jnp.sin(angle)
    out_ref[0, 0:_TIME, :] = jnp.broadcast_to(
        pe[:, None, :], (_TIME, _K, _L)
    ).reshape(_TIME, _KL)
    out_ref[0, _TIME:_C, :] = jnp.broadcast_to(
        ett_ref[...][:, :, None], (_FEAT, _K, _L)
    ).reshape(_FEAT, _KL)


def kernel(observed_tp, observed_mask, embed_table):
    del observed_mask
    half = jnp.power(
        10000.0, -jnp.arange(0, _TIME, 2, dtype=jnp.float32) / _TIME
    )
    div2 = jnp.repeat(half, 2).reshape(_TIME, 1)
    off = jnp.tile(jnp.array([0.0, np.pi / 2], jnp.float32), _TIME // 2)
    off = off.reshape(_TIME, 1)
    ett = embed_table.T
    tp3 = observed_tp.reshape(_B, 1, _L)

    out = pl.pallas_call(
        _body,
        grid=(_B,),
        in_specs=[
            pl.BlockSpec((1, 1, _L), lambda b: (b, 0, 0)),
            pl.BlockSpec((_TIME, 1), lambda b: (0, 0)),
            pl.BlockSpec((_TIME, 1), lambda b: (0, 0)),
            pl.BlockSpec((_FEAT, _K), lambda b: (0, 0)),
        ],
        out_specs=pl.BlockSpec((1, _C, _KL), lambda b: (b, 0, 0)),
        out_shape=jax.ShapeDtypeStruct((_B, _C, _KL), jnp.float32),
    )(tp3, div2, off, ett)
    return out.reshape(_B, _C, _K, _L)


# speedup vs baseline: 1.5976x; 1.5976x over previous
import functools

import jax
import jax.numpy as jnp
import numpy as np
from jax import lax
from jax.experimental import pallas as pl
from jax.experimental.pallas import tpu as pltpu
from jax.experimental.pallas import tpu_sc as plsc

_B, _K, _L = 16, 100, 200
_TIME = 128
_FEAT = 16
_C = _TIME + _FEAT
_NW = 32
_PLANES = _B * _C

_ROW_OFFS = tuple(range(0, 192, 16)) + (184,)

_S3, _S5, _S7, _S9, _S11 = (
    -1.0 / 6.0,
    1.0 / 120.0,
    -1.0 / 5040.0,
    1.0 / 362880.0,
    -1.0 / 39916800.0,
)


def _sin(x):
    x2 = x * x
    p = _S9 + x2 * _S11
    p = _S7 + x2 * p
    p = _S5 + x2 * p
    p = _S3 + x2 * p
    return x + x * x2 * p


def _sc_body(tp_hbm, et_hbm, d2_hbm, off_hbm, out_hbm, tp_v, et_v, d2_v, off_v, plane_v):
    wid = lax.axis_index("s") * 2 + lax.axis_index("c")
    pltpu.sync_copy(tp_hbm, tp_v)
    pltpu.sync_copy(et_hbm, et_v)
    pltpu.sync_copy(d2_hbm, d2_v)
    pltpu.sync_copy(off_hbm, off_v)

    def do_plane(i, carry):
        p = i * _NW + wid
        b = p // _C
        c = p % _C

        def build_time():
            ci = jnp.full((16,), c, jnp.int32)
            d2 = plsc.load_gather(d2_v, [ci])
            ph = plsc.load_gather(off_v, [ci])
            vecs = []
            for off in _ROW_OFFS:
                x = tp_v[pl.ds(b * _L + off, 16)] * d2 + ph
                vecs.append(_sin(x))

            def fill_row(k, _):
                for off, v in zip(_ROW_OFFS, vecs):
                    plane_v[k, pl.ds(off, 16)] = v
                return 0

            lax.fori_loop(0, _K, fill_row, 0)

        def build_feat():
            f = c - _TIME

            def fill_row(k, _):
                v = plsc.load_gather(
                    et_v, [jnp.full((16,), k * _FEAT + f, jnp.int32)]
                )
                for off in _ROW_OFFS:
                    plane_v[k, pl.ds(off, 16)] = v
                return 0

            lax.fori_loop(0, _K, fill_row, 0)

        lax.cond(c < _TIME, build_time, build_feat)
        pltpu.sync_copy(plane_v, out_hbm.at[b, c])
        return carry

    lax.fori_loop(0, _PLANES // _NW, do_plane, 0)


def kernel(observed_tp, observed_mask, embed_table):
    del observed_mask
    half = jnp.power(
        10000.0, -jnp.arange(0, _TIME, 2, dtype=jnp.float32) / _TIME
    )
    div2 = jnp.repeat(half, 2)
    off = jnp.tile(jnp.array([0.0, np.pi / 2], jnp.float32), _TIME // 2)

    mesh = plsc.VectorSubcoreMesh(core_axis_name="c", subcore_axis_name="s")
    run = pl.kernel(
        _sc_body,
        out_type=jax.ShapeDtypeStruct((_B, _C, _K, _L), jnp.float32),
        mesh=mesh,
        compiler_params=pltpu.CompilerParams(needs_layout_passes=False),
        scratch_types=[
            pltpu.VMEM((_B * _L,), jnp.float32),
            pltpu.VMEM((_K * _FEAT,), jnp.float32),
            pltpu.VMEM((_TIME,), jnp.float32),
            pltpu.VMEM((_TIME,), jnp.float32),
            pltpu.VMEM((_K, _L), jnp.float32),
        ],
    )
    return run(
        observed_tp.reshape(-1),
        embed_table.reshape(-1),
        div2,
        off,
    )
